# Initial kernel scaffold; baseline (speedup 1.0000x reference)
import functools

import jax
import jax.numpy as jnp
from jax import lax
from jax.experimental import pallas as pl
from jax.experimental.pallas import tpu as pltpu
from jax.experimental.pallas import tpu_sc as plsc

N, P2, TOPK, W2, CKV = 8, 49, 4, 64, 384
NW = 32
ITEMS = N * P2 * TOPK
IPW = ITEMS // NW
LANES = 16
CHUNKS = CKV // LANES


def _body(bidx_hbm, w_hbm, kv_hbm, out_hbm, bidx_v, w_v, buf):
    nc = 2
    wid = lax.axis_index("s") * nc + lax.axis_index("c")

    pltpu.sync_copy(bidx_hbm.at[wid], bidx_v)
    pltpu.sync_copy(w_hbm.at[wid], w_v)

    def item(t, carry):
        base = jnp.max(bidx_v[t])
        pltpu.sync_copy(kv_hbm.at[base], buf)
        wv = w_v[t]

        def row(r, c2):
            for cc in range(CHUNKS):
                sl = pl.ds(cc * LANES, LANES)
                buf[r, sl] = buf[r, sl] * wv
            return c2

        lax.fori_loop(0, W2, row, 0)
        pltpu.sync_copy(buf, out_hbm.at[wid * IPW + t])
        return carry

    lax.fori_loop(0, IPW, item, 0)


@functools.partial(
    pl.kernel,
    mesh=plsc.VectorSubcoreMesh(core_axis_name="c", subcore_axis_name="s"),
    out_type=jax.ShapeDtypeStruct((ITEMS, W2, CKV), jnp.float32),
    scratch_types=[
        pltpu.VMEM((IPW, LANES), jnp.int32),
        pltpu.VMEM((IPW, LANES), jnp.float32),
        pltpu.VMEM((W2, CKV), jnp.float32),
    ],
)
def _gather_scale(bidx_hbm, w_hbm, kv_hbm, out_hbm, bidx_v, w_v, buf):
    _body(bidx_hbm, w_hbm, kv_hbm, out_hbm, bidx_v, w_v, buf)


def kernel(r_idx, r_weight, kv):
    n, p2, w2, c_kv = kv.shape
    topk = r_idx.shape[-1]
    base = (jnp.arange(n, dtype=jnp.int32)[:, None, None] * p2
            + r_idx.astype(jnp.int32)).reshape(NW, IPW)
    bidx = jnp.broadcast_to(base[:, :, None], (NW, IPW, LANES))
    wgt = jnp.broadcast_to(
        r_weight.astype(jnp.float32).reshape(NW, IPW)[:, :, None],
        (NW, IPW, LANES))
    kvr = kv.reshape(n * p2, w2, c_kv)
    out = _gather_scale(bidx, wgt, kvr)
    return out.reshape(n, p2, topk, w2, c_kv)


# speedup vs baseline: 1.0841x; 1.0841x over previous
import functools

import jax
import jax.numpy as jnp
from jax import lax
from jax.experimental import pallas as pl
from jax.experimental.pallas import tpu as pltpu
from jax.experimental.pallas import tpu_sc as plsc

N, P2, TOPK, W2, CKV = 8, 49, 4, 64, 384
NW = 32
ITEMS = N * P2 * TOPK
IPW = ITEMS // NW
LANES = 16
CHUNKS = CKV // LANES


def _body(bidx_hbm, w_hbm, kv_hbm, out_hbm, bidx_v, w_v, buf):
    nc = 2
    wid = lax.axis_index("s") * nc + lax.axis_index("c")

    pltpu.sync_copy(bidx_hbm.at[wid], bidx_v)
    pltpu.sync_copy(w_hbm.at[wid], w_v)

    def item(t, carry):
        base = jnp.max(bidx_v[t])
        pltpu.sync_copy(kv_hbm.at[base], buf)
        wv = w_v[t]

        def row(r, c2):
            for cc in range(CHUNKS):
                sl = pl.ds(cc * LANES, LANES)
                buf[r, sl] = buf[r, sl] * wv
            return c2

        lax.fori_loop(0, W2, row, 0)
        pltpu.sync_copy(buf, out_hbm.at[wid * IPW + t])
        return carry

    lax.fori_loop(0, IPW, item, 0)


@functools.partial(
    pl.kernel,
    mesh=plsc.VectorSubcoreMesh(core_axis_name="c", subcore_axis_name="s"),
    out_type=jax.ShapeDtypeStruct((ITEMS, W2, CKV), jnp.float32),
    scratch_types=[
        pltpu.VMEM((IPW, LANES), jnp.int32),
        pltpu.VMEM((IPW, LANES), jnp.float32),
        pltpu.VMEM((W2, CKV), jnp.float32),
    ],
    compiler_params=pltpu.CompilerParams(needs_layout_passes=False),
)
def _gather_scale(bidx_hbm, w_hbm, kv_hbm, out_hbm, bidx_v, w_v, buf):
    _body(bidx_hbm, w_hbm, kv_hbm, out_hbm, bidx_v, w_v, buf)


def kernel(r_idx, r_weight, kv):
    n, p2, w2, c_kv = kv.shape
    topk = r_idx.shape[-1]
    base = (jnp.arange(n, dtype=jnp.int32)[:, None, None] * p2
            + r_idx.astype(jnp.int32)).reshape(NW, IPW)
    bidx = jnp.broadcast_to(base[:, :, None], (NW, IPW, LANES))
    wgt = jnp.broadcast_to(
        r_weight.astype(jnp.float32).reshape(NW, IPW)[:, :, None],
        (NW, IPW, LANES))
    kvr = kv.reshape(n * p2, w2, c_kv)
    out = _gather_scale(bidx, wgt, kvr)
    return out.reshape(n, p2, topk, w2, c_kv)


# speedup vs baseline: 1.7041x; 1.5718x over previous
import functools

import jax
import jax.numpy as jnp
from jax import lax
from jax.experimental import pallas as pl
from jax.experimental.pallas import tpu as pltpu
from jax.experimental.pallas import tpu_sc as plsc

N, P2, TOPK, W2, CKV = 8, 49, 4, 64, 384
NW = 32
ITEMS = N * P2 * TOPK
IPW = ITEMS // NW
LANES = 16
CHUNKS = CKV // LANES
NBUF = 4
GROUPS = (IPW - 1) // NBUF


def _body(bidx_hbm, w_hbm, kv_hbm, out_hbm, bidx_v, w_v, buf,
          in_sems, out_sems):
    nc = 2
    wid = lax.axis_index("s") * nc + lax.axis_index("c")

    pltpu.sync_copy(bidx_hbm.at[wid], bidx_v)
    pltpu.sync_copy(w_hbm.at[wid], w_v)

    def start_gather(item, slot):
        base = jnp.max(bidx_v[item])
        pltpu.async_copy(kv_hbm.at[base], buf.at[slot], in_sems.at[slot])

    def wait_in(slot):
        pltpu.make_async_copy(kv_hbm.at[0], buf.at[slot],
                              in_sems.at[slot]).wait()

    def start_out(item, slot):
        pltpu.async_copy(buf.at[slot], out_hbm.at[wid * IPW + item],
                         out_sems.at[slot])

    def wait_out(slot):
        pltpu.make_async_copy(buf.at[slot], out_hbm.at[0],
                              out_sems.at[slot]).wait()

    def scale(item, slot):
        wv = w_v[item]

        def row(r, c2):
            for cc in range(CHUNKS):
                sl = pl.ds(cc * LANES, LANES)
                buf[slot, r, sl] = buf[slot, r, sl] * wv
            return c2

        lax.fori_loop(0, W2, row, 0)

    start_gather(0, 0)
    start_gather(1, 1)

    def group(g, carry):
        for b in range(NBUF):
            j = g * NBUF + b
            b2 = (b + 2) % NBUF
            @pl.when(j >= 2)
            def _():
                wait_out(b2)

            @pl.when(j + 2 < IPW)
            def _():
                start_gather(j + 2, b2)

            wait_in(b)
            scale(j, b)
            start_out(j, b)
        return carry

    lax.fori_loop(0, GROUPS, group, 0)

    last = IPW - 1
    wait_in(last % NBUF)
    scale(last, last % NBUF)
    start_out(last, last % NBUF)

    wait_out(2)
    wait_out(3)
    wait_out(0)


@functools.partial(
    pl.kernel,
    mesh=plsc.VectorSubcoreMesh(core_axis_name="c", subcore_axis_name="s"),
    out_type=jax.ShapeDtypeStruct((ITEMS, W2, CKV), jnp.float32),
    scratch_types=[
        pltpu.VMEM((IPW, LANES), jnp.int32),
        pltpu.VMEM((IPW, LANES), jnp.float32),
        pltpu.VMEM((NBUF, W2, CKV), jnp.float32),
        pltpu.SemaphoreType.DMA((NBUF,)),
        pltpu.SemaphoreType.DMA((NBUF,)),
    ],
    compiler_params=pltpu.CompilerParams(needs_layout_passes=False),
)
def _gather_scale(bidx_hbm, w_hbm, kv_hbm, out_hbm, bidx_v, w_v, buf,
                  in_sems, out_sems):
    _body(bidx_hbm, w_hbm, kv_hbm, out_hbm, bidx_v, w_v, buf,
          in_sems, out_sems)


def kernel(r_idx, r_weight, kv):
    n, p2, w2, c_kv = kv.shape
    topk = r_idx.shape[-1]
    base = (jnp.arange(n, dtype=jnp.int32)[:, None, None] * p2
            + r_idx.astype(jnp.int32)).reshape(NW, IPW)
    bidx = jnp.broadcast_to(base[:, :, None], (NW, IPW, LANES))
    wgt = jnp.broadcast_to(
        r_weight.astype(jnp.float32).reshape(NW, IPW)[:, :, None],
        (NW, IPW, LANES))
    kvr = kv.reshape(n * p2, w2, c_kv)
    out = _gather_scale(bidx, wgt, kvr)
    return out.reshape(n, p2, topk, w2, c_kv)
